# baseline (speedup 1.0000x reference)
import jax
import jax.numpy as jnp
from jax import lax
from jax.experimental import pallas as pl
from jax.experimental.pallas import tpu as pltpu
from jax.experimental.pallas import tpu_sc as plsc

N_NODES = 10000
N_EDGES = 320000
HID = 128

NC = 2
NS = 16
NW = NC * NS
CHUNK = 128
N_CHUNKS_PAD = 2560
CPW = N_CHUNKS_PAD // NW
N_EDGES_PAD = N_CHUNKS_PAD * CHUNK
TBL_ROWS = N_NODES + 8
DUMMY = N_NODES


def _k1_body(nf, w0a, w0b, pa, pb):
    x = nf[...]
    pa[...] = jnp.dot(x, w0a[...], preferred_element_type=jnp.float32)
    pb[...] = jnp.dot(x, w0b[...], preferred_element_type=jnp.float32)


def _project(nf, w0a, w0b):
    B = 1000
    grid = N_NODES // B
    return pl.pallas_call(
        _k1_body,
        grid=(grid,),
        in_specs=[
            pl.BlockSpec((B, HID), lambda i: (i, 0)),
            pl.BlockSpec((HID, HID), lambda i: (0, 0)),
            pl.BlockSpec((HID, HID), lambda i: (0, 0)),
        ],
        out_specs=[
            pl.BlockSpec((B, HID), lambda i: (i, 0)),
            pl.BlockSpec((B, HID), lambda i: (i, 0)),
        ],
        out_shape=[
            jax.ShapeDtypeStruct((TBL_ROWS, HID), jnp.float32),
            jax.ShapeDtypeStruct((TBL_ROWS, HID), jnp.float32),
        ],
    )(nf, w0a, w0b)


def _k2_body(sidx_hbm, ridx_hbm, pa_hbm, pb_hbm, ga_hbm, gb_hbm,
             sidx_v, ridx_v,
             rows_a0, rows_a1, rows_b0, rows_b1,
             gsa0, gsa1, gsb0, gsb1, ssa0, ssa1, ssb0, ssb1):
    wid = lax.axis_index("s") * NC + lax.axis_index("c")
    base = wid * CPW
    rows_a = (rows_a0, rows_a1)
    rows_b = (rows_b0, rows_b1)
    gsa = (gsa0, gsa1)
    gsb = (gsb0, gsb1)
    ssa = (ssa0, ssa1)
    ssb = (ssb0, ssb1)

    pltpu.sync_copy(sidx_hbm.at[pl.ds(base, CPW)], sidx_v)
    pltpu.sync_copy(ridx_hbm.at[pl.ds(base, CPW)], ridx_v)

    for b in range(2):
        pltpu.async_copy(pa_hbm.at[sidx_v.at[b]], rows_a[b], gsa[b])
        pltpu.async_copy(pb_hbm.at[ridx_v.at[b]], rows_b[b], gsb[b])

    def body(i, carry):
        for b in range(2):
            t = 2 * i + b
            pltpu.make_async_copy(ga_hbm.at[base], rows_a[b], gsa[b]).wait()
            pltpu.make_async_copy(gb_hbm.at[base], rows_b[b], gsb[b]).wait()
            pltpu.async_copy(rows_a[b], ga_hbm.at[base + t], ssa[b])
            pltpu.async_copy(rows_b[b], gb_hbm.at[base + t], ssb[b])
        for b in range(2):
            t = 2 * i + b + 2

            @pl.when(t < CPW)
            def _(b=b, t=t):
                pltpu.make_async_copy(rows_a[b], ga_hbm.at[base], ssa[b]).wait()
                pltpu.make_async_copy(rows_b[b], gb_hbm.at[base], ssb[b]).wait()
                pltpu.async_copy(pa_hbm.at[sidx_v.at[t]], rows_a[b], gsa[b])
                pltpu.async_copy(pb_hbm.at[ridx_v.at[t]], rows_b[b], gsb[b])

        return carry

    lax.fori_loop(0, CPW // 2, body, 0)
    for b in range(2):
        pltpu.make_async_copy(rows_a[b], ga_hbm.at[base], ssa[b]).wait()
        pltpu.make_async_copy(rows_b[b], gb_hbm.at[base], ssb[b]).wait()


def _gather(sidx, ridx, pa, pb):
    mesh = plsc.VectorSubcoreMesh(core_axis_name="c", subcore_axis_name="s")
    f = pl.kernel(
        _k2_body,
        out_type=[
            jax.ShapeDtypeStruct((N_CHUNKS_PAD, CHUNK, HID), jnp.float32),
            jax.ShapeDtypeStruct((N_CHUNKS_PAD, CHUNK, HID), jnp.float32),
        ],
        mesh=mesh,
        scratch_types=[
            pltpu.VMEM((CPW, CHUNK), jnp.int32),
            pltpu.VMEM((CPW, CHUNK), jnp.int32),
            pltpu.VMEM((CHUNK, HID), jnp.float32),
            pltpu.VMEM((CHUNK, HID), jnp.float32),
            pltpu.VMEM((CHUNK, HID), jnp.float32),
            pltpu.VMEM((CHUNK, HID), jnp.float32),
        ] + [pltpu.SemaphoreType.DMA] * 8,
    )
    return f(sidx, ridx, pa, pb)


def _k3_body(ga, gb, ef, w0c, b0, w1, b1, w2, b2, g, beta, pre, res):
    e = ef[...]
    h = ga[...] + gb[...] + jnp.dot(e, w0c[...], preferred_element_type=jnp.float32)
    h = jnp.maximum(h + b0[...], 0.0)
    h = jnp.maximum(jnp.dot(h, w1[...], preferred_element_type=jnp.float32) + b1[...], 0.0)
    h = jnp.dot(h, w2[...], preferred_element_type=jnp.float32) + b2[...]
    mu = jnp.mean(h, axis=-1, keepdims=True)
    d = h - mu
    var = jnp.mean(d * d, axis=-1, keepdims=True)
    ln = g[...] * d * lax.rsqrt(var + 1e-5) + beta[...]
    pre[...] = ln
    res[...] = ln + e


def _edge_mlp(ga, gb, ef, w0c, b0, w1, b1, w2, b2, g, beta):
    B = 2000
    grid = N_EDGES // B
    wspec = pl.BlockSpec((HID, HID), lambda i: (0, 0))
    vspec = pl.BlockSpec((1, HID), lambda i: (0, 0))
    rspec = pl.BlockSpec((B, HID), lambda i: (i, 0))
    return pl.pallas_call(
        _k3_body,
        grid=(grid,),
        in_specs=[rspec, rspec, rspec, wspec, vspec, wspec, vspec, wspec,
                  vspec, vspec, vspec],
        out_specs=[rspec, rspec],
        out_shape=[
            jax.ShapeDtypeStruct((N_EDGES_PAD, HID), jnp.float32),
            jax.ShapeDtypeStruct((N_EDGES, HID), jnp.float32),
        ],
    )(ga, gb, ef, w0c, b0, w1, b1, w2, b2, g, beta)


def _k4_body(ridx_hbm, pre_hbm, zeros_hbm, out_hbm,
             ridx_v, rows0, rows1, ls0, ls1, acc):
    cid = lax.axis_index("c")
    sid = lax.axis_index("s")
    base = (sid * NC + cid) * CPW
    rows = (rows0, rows1)
    ls = (ls0, ls1)

    pltpu.sync_copy(ridx_hbm.at[pl.ds(base, CPW)], ridx_v)

    @pl.when(sid == 0)
    def _():
        pltpu.sync_copy(zeros_hbm, acc)

    plsc.subcore_barrier()

    pltpu.async_copy(pre_hbm.at[base], rows[0], ls[0])

    def body(i, carry):
        for b in range(2):
            t = 2 * i + b
            pltpu.make_async_copy(pre_hbm.at[base], rows[b], ls[b]).wait()

            @pl.when(t + 1 < CPW)
            def _(b=b, t=t):
                pltpu.async_copy(pre_hbm.at[base + t + 1], rows[1 - b], ls[1 - b])

            pltpu.sync_copy(rows[b], acc.at[ridx_v.at[t]], add=True)
        return carry

    lax.fori_loop(0, CPW // 2, body, 0)
    plsc.subcore_barrier()

    @pl.when(sid == 0)
    def _():
        pltpu.sync_copy(acc.at[pl.ds(0, N_NODES)], out_hbm.at[cid])


def _scatter(ridx, pre, zeros):
    mesh = plsc.VectorSubcoreMesh(core_axis_name="c", subcore_axis_name="s")
    f = pl.kernel(
        _k4_body,
        out_type=jax.ShapeDtypeStruct((NC, N_NODES, HID), jnp.float32),
        mesh=mesh,
        scratch_types=[
            pltpu.VMEM((CPW, CHUNK), jnp.int32),
            pltpu.VMEM((CHUNK, HID), jnp.float32),
            pltpu.VMEM((CHUNK, HID), jnp.float32),
            pltpu.SemaphoreType.DMA,
            pltpu.SemaphoreType.DMA,
            pltpu.VMEM_SHARED((TBL_ROWS, HID), jnp.float32),
        ],
    )
    return f(ridx, pre, zeros)


def _k5_body(nf, p0, p1, w0a, w0b, b0, w1, b1, w2, b2, g, beta, out):
    x = nf[...]
    a = p0[0] + p1[0]
    h = (jnp.dot(x, w0a[...], preferred_element_type=jnp.float32)
         + jnp.dot(a, w0b[...], preferred_element_type=jnp.float32))
    h = jnp.maximum(h + b0[...], 0.0)
    h = jnp.maximum(jnp.dot(h, w1[...], preferred_element_type=jnp.float32) + b1[...], 0.0)
    h = jnp.dot(h, w2[...], preferred_element_type=jnp.float32) + b2[...]
    mu = jnp.mean(h, axis=-1, keepdims=True)
    d = h - mu
    var = jnp.mean(d * d, axis=-1, keepdims=True)
    out[...] = g[...] * d * lax.rsqrt(var + 1e-5) + beta[...] + x


def _node_mlp(nf, partials, w0a, w0b, b0, w1, b1, w2, b2, g, beta):
    B = 1000
    grid = N_NODES // B
    wspec = pl.BlockSpec((HID, HID), lambda i: (0, 0))
    vspec = pl.BlockSpec((1, HID), lambda i: (0, 0))
    rspec = pl.BlockSpec((B, HID), lambda i: (i, 0))
    return pl.pallas_call(
        _k5_body,
        grid=(grid,),
        in_specs=[
            rspec,
            pl.BlockSpec((1, B, HID), lambda i: (0, i, 0)),
            pl.BlockSpec((1, B, HID), lambda i: (1, i, 0)),
            wspec, wspec, vspec, wspec, vspec, wspec, vspec, vspec, vspec,
        ],
        out_specs=rspec,
        out_shape=jax.ShapeDtypeStruct((N_NODES, HID), jnp.float32),
    )(nf, partials, partials, w0a, w0b, b0, w1, b1, w2, b2, g, beta)


def kernel(edge_idx, node_features, edge_features,
           e_W0, e_b0, e_W1, e_b1, e_W2, e_b2, e_g, e_beta,
           n_W0, n_b0, n_W1, n_b1, n_W2, n_b2, n_g, n_beta):
    ei = edge_idx.astype(jnp.int32)
    pad = jnp.full((2, N_EDGES_PAD - N_EDGES), DUMMY, jnp.int32)
    ei = jnp.concatenate([ei, pad], axis=1)
    senders = ei[0].reshape(N_CHUNKS_PAD, CHUNK)
    receivers = ei[1].reshape(N_CHUNKS_PAD, CHUNK)

    e_w0a = e_W0[:HID]
    e_w0b = e_W0[HID:2 * HID]
    e_w0c = e_W0[2 * HID:]
    n_w0a = n_W0[:HID]
    n_w0b = n_W0[HID:]

    r1 = lambda v: v.reshape(1, HID)

    pa, pb = _project(node_features, e_w0a, e_w0b)
    ga, gb = _gather(senders, receivers, pa, pb)
    ga = ga.reshape(N_EDGES_PAD, HID)
    gb = gb.reshape(N_EDGES_PAD, HID)
    pre, new_edge = _edge_mlp(ga, gb, edge_features, e_w0c, r1(e_b0),
                              e_W1, r1(e_b1), e_W2, r1(e_b2),
                              r1(e_g), r1(e_beta))
    zeros = jnp.zeros((TBL_ROWS, HID), jnp.float32)
    partials = _scatter(receivers, pre.reshape(N_CHUNKS_PAD, CHUNK, HID), zeros)
    new_node = _node_mlp(node_features, partials, n_w0a, n_w0b, r1(n_b0),
                         n_W1, r1(n_b1), n_W2, r1(n_b2), r1(n_g), r1(n_beta))
    return (new_node, new_edge)
